# baseline (speedup 1.0000x reference)
import functools

import jax
import jax.numpy as jnp
from jax import lax
from jax.experimental import pallas as pl
from jax.experimental.pallas import tpu as pltpu
from jax.experimental.pallas import tpu_sc as plsc

_N = 10000
_E = 320000
_D = 128
_H = 32

_NC = 2
_NS = 16
_NW = _NC * _NS
_K = 128
_NPAD = 10240
_EPAD = 327680
_PW = _EPAD // _NW
_KCH = _PW // _K
_RPT = _NPAD // _NS

def _mesh():
    return plsc.VectorSubcoreMesh(
        core_axis_name="c", subcore_axis_name="s",
        num_cores=_NC, num_subcores=_NS)


_U = 8
_G = _KCH // _U
_NBUF = 4
_KCH0 = 56
_KCH1 = 2 * _KCH - _KCH0


@functools.cache
def _sc_gather_fn():
    return functools.partial(
        pl.kernel,
        out_type=(
            jax.ShapeDtypeStruct((_EPAD, 64), jnp.float32),
            jax.ShapeDtypeStruct((_EPAD, 32), jnp.float32),
        ),
        mesh=_mesh(),
        compiler_params=pltpu.CompilerParams(use_tc_tiling_on_sc=False),
        scratch_types=[
            pltpu.VMEM((_KCH1, _K), jnp.int32),
            pltpu.VMEM((_KCH1, _K), jnp.int32),
        ] + [pltpu.VMEM((_K, 64), jnp.float32) for _ in range(_NBUF)]
          + [pltpu.VMEM((_K, 32), jnp.float32) for _ in range(_NBUF)]
          + [pltpu.SemaphoreType.DMA for _ in range(2 * _NBUF)],
    )(_sc_gather_body)


def _sc_gather(su, t, src2d, dst2d):
    return _sc_gather_fn()(su, t, src2d, dst2d)


def _sc_gather_body(su_hbm, t_hbm, src_hbm, dst_hbm, gs_hbm, gd_hbm,
                    sidx, didx, *bufsem):
    c = lax.axis_index("c")
    s = lax.axis_index("s")
    gsb = bufsem[0:_NBUF]
    gdb = bufsem[_NBUF:2 * _NBUF]
    gsem = bufsem[2 * _NBUF:3 * _NBUF]
    ssem = bufsem[3 * _NBUF:4 * _NBUF]

    def gath(base, j, b):
        return (pltpu.async_copy(su_hbm.at[sidx.at[j]], gsb[b], gsem[b]),
                pltpu.async_copy(t_hbm.at[didx.at[j]], gdb[b], gsem[b]))

    def stor(base, j, b):
        return (pltpu.async_copy(gsb[b], gs_hbm.at[pl.ds(base + j * _K, _K)],
                                 ssem[b]),
                pltpu.async_copy(gdb[b], gd_hbm.at[pl.ds(base + j * _K, _K)],
                                 ssem[b]))

    def run(start_chunk, kch):
        pltpu.sync_copy(src_hbm.at[pl.ds(start_chunk, kch)],
                        sidx.at[pl.ds(0, kch)])
        pltpu.sync_copy(dst_hbm.at[pl.ds(start_chunk, kch)],
                        didx.at[pl.ds(0, kch)])
        base = start_chunk * _K

        def group(g, carry):
            c0 = g * _U
            gd_ = [None] * _U
            st = [None] * _U
            for b in range(_U):
                if b >= _NBUF:
                    st[b - _NBUF][0].wait()
                    st[b - _NBUF][1].wait()
                gd_[b] = gath(base, c0 + b, b % _NBUF)
                ib = b - (_NBUF - 1)
                if ib >= 0:
                    gd_[ib][0].wait()
                    gd_[ib][1].wait()
                    st[ib] = stor(base, c0 + ib, ib % _NBUF)
            for ib in range(_U - _NBUF + 1, _U):
                gd_[ib][0].wait()
                gd_[ib][1].wait()
                st[ib] = stor(base, c0 + ib, ib % _NBUF)
            for ib in range(_U - _NBUF, _U):
                st[ib][0].wait()
                st[ib][1].wait()
            return carry

        lax.fori_loop(0, kch // _U, group, 0)

    start = s * (2 * _KCH) + c * _KCH0

    @pl.when(c == 0)
    def _():
        run(start, _KCH0)

    @pl.when(c == 1)
    def _():
        run(start, _KCH1)


@functools.cache
def _sc_scatter_fn():
    return functools.partial(
        pl.kernel,
        out_type=(
            jax.ShapeDtypeStruct((_NPAD, 32), jnp.float32),
            jax.ShapeDtypeStruct((_NPAD, 32), jnp.float32),
        ),
        mesh=_mesh(),
        compiler_params=pltpu.CompilerParams(use_tc_tiling_on_sc=False),
        scratch_types=[
            pltpu.VMEM((_KCH, _K), jnp.int32),
            pltpu.VMEM((_K, 32), jnp.float32),
            pltpu.VMEM((_K, 32), jnp.float32),
            pltpu.VMEM_SHARED((_NPAD, 32), jnp.float32),
            pltpu.SemaphoreType.DMA,
            pltpu.SemaphoreType.DMA,
        ],
    )(_sc_scatter_body)


def _sc_scatter(vals, dst2d):
    return _sc_scatter_fn()(vals, dst2d)


def _zero_vbuf(vbuf, val=0.0):
    zv = jnp.full((16,), val, jnp.float32)

    def zrow(i, carry):
        vbuf[i, pl.ds(0, 16)] = zv
        vbuf[i, pl.ds(16, 16)] = zv
        return carry

    lax.fori_loop(0, _K, zrow, 0)


def _sc_scatter_body(vals_hbm, dst_hbm, out0_hbm, out1_hbm,
                     didx, vbuf0, vbuf1, shared, sem0, sem1):
    c = lax.axis_index("c")
    s = lax.axis_index("s")
    wid = s * _NC + c
    base = wid * _PW
    pltpu.sync_copy(dst_hbm.at[pl.ds(wid * _KCH, _KCH)], didx)

    _zero_vbuf(vbuf0)
    for i in range(_RPT // _K):
        pltpu.sync_copy(vbuf0, shared.at[pl.ds(s * _RPT + i * _K, _K)])
    plsc.subcore_barrier()

    bufs = ((vbuf0, sem0), (vbuf1, sem1))

    def group(g, carry):
        c0 = g * _U
        vb, sm = bufs[0]
        pend = pltpu.async_copy(vals_hbm.at[pl.ds(base + c0 * _K, _K)], vb, sm)
        for b in range(_U):
            j = c0 + b
            vb, _ = bufs[b % 2]
            nvb, nsm = bufs[(b + 1) % 2]
            nxt = None
            if b + 1 < _U:
                nxt = pltpu.async_copy(
                    vals_hbm.at[pl.ds(base + (j + 1) * _K, _K)], nvb, nsm)
            pend.wait()
            pltpu.sync_copy(vb, shared.at[didx.at[j]], add=True)
            pend = nxt
        return carry

    lax.fori_loop(0, _G, group, 0)
    plsc.subcore_barrier()

    @pl.when(c == 0)
    def _():
        pltpu.sync_copy(shared.at[pl.ds(s * _RPT, _RPT)],
                        out0_hbm.at[pl.ds(s * _RPT, _RPT)])

    @pl.when(c == 1)
    def _():
        pltpu.sync_copy(shared.at[pl.ds(s * _RPT, _RPT)],
                        out1_hbm.at[pl.ds(s * _RPT, _RPT)])


@functools.cache
def _sc_count_fn():
    return functools.partial(
        pl.kernel,
        out_type=(
            jax.ShapeDtypeStruct((_NPAD, 32), jnp.float32),
            jax.ShapeDtypeStruct((_NPAD, 32), jnp.float32),
        ),
        mesh=_mesh(),
        compiler_params=pltpu.CompilerParams(use_tc_tiling_on_sc=False),
        scratch_types=[
            pltpu.VMEM((_KCH, _K), jnp.int32),
            pltpu.VMEM((_K, 32), jnp.float32),
            pltpu.VMEM_SHARED((_NPAD, 32), jnp.float32),
        ],
    )(_sc_count_body)


def _sc_count(dst2d):
    return _sc_count_fn()(dst2d)


def _sc_count_body(dst_hbm, out0_hbm, out1_hbm, didx, vbuf, shared):
    c = lax.axis_index("c")
    s = lax.axis_index("s")
    wid = s * _NC + c
    pltpu.sync_copy(dst_hbm.at[pl.ds(wid * _KCH, _KCH)], didx)

    _zero_vbuf(vbuf)
    for i in range(_RPT // _K):
        pltpu.sync_copy(vbuf, shared.at[pl.ds(s * _RPT + i * _K, _K)])
    plsc.subcore_barrier()
    _zero_vbuf(vbuf, 1.0)

    def step(j, carry):
        pltpu.sync_copy(vbuf, shared.at[didx.at[j]], add=True)
        return carry

    lax.fori_loop(0, _KCH, step, 0)
    plsc.subcore_barrier()

    @pl.when(c == 0)
    def _():
        pltpu.sync_copy(shared.at[pl.ds(s * _RPT, _RPT)],
                        out0_hbm.at[pl.ds(s * _RPT, _RPT)])

    @pl.when(c == 1)
    def _():
        pltpu.sync_copy(shared.at[pl.ds(s * _RPT, _RPT)],
                        out1_hbm.at[pl.ds(s * _RPT, _RPT)])


def _lrelu_ln(h, g, b):
    a = jnp.where(h >= 0, h, 0.01 * h)
    m = jnp.mean(a, axis=-1, keepdims=True)
    v = jnp.mean((a - m) ** 2, axis=-1, keepdims=True)
    return (a - m) * lax.rsqrt(v + 1e-5) * g + b


def _dot(a, b):
    return jnp.dot(a, b, preferred_element_type=jnp.float32)


def _tables_body(x_ref, wsu_ref, wt_ref, su_ref, t_ref):
    x = x_ref[...]
    su_ref[...] = _dot(x, wsu_ref[...])
    t_ref[...] = _dot(x, wt_ref[...])


def _tc_tables(x, wsu, wt):
    n, d = x.shape
    blk = 2048
    return pl.pallas_call(
        _tables_body,
        grid=(n // blk,),
        in_specs=[
            pl.BlockSpec((blk, d), lambda i: (i, 0)),
            pl.BlockSpec((d, 64), lambda i: (0, 0)),
            pl.BlockSpec((d, 32), lambda i: (0, 0)),
        ],
        out_specs=(
            pl.BlockSpec((blk, 64), lambda i: (i, 0)),
            pl.BlockSpec((blk, 32), lambda i: (i, 0)),
        ),
        out_shape=(
            jax.ShapeDtypeStruct((n, 64), jnp.float32),
            jax.ShapeDtypeStruct((n, 32), jnp.float32),
        ),
    )(x, wsu, wt)


def _lrelu_ln4(h, av, g, b):
    a = jnp.where(h >= 0, h, 0.01 * h)
    m = _dot(a, av)
    d = a - m
    v = _dot(d * d, av)
    return d * lax.rsqrt(v + 1e-5) * g + b


def _edge_body(gs_ref, gd_ref, ea_ref, we_ref, mats_ref, av_ref, vecs_ref,
               ne_ref, no_ref):
    x = gs_ref[...]
    s4 = jnp.concatenate(
        [x[:, 0:32], x[:, 64:96], x[:, 128:160], x[:, 192:224]], axis=1)
    u4 = jnp.concatenate(
        [x[:, 32:64], x[:, 96:128], x[:, 160:192], x[:, 224:256]], axis=1)
    av = av_ref[...]
    mats = mats_ref[...]
    vecs = vecs_ref[...]
    eh = s4 + gd_ref[...] + _dot(ea_ref[...], we_ref[...]) + vecs[0:1]
    eh = _lrelu_ln4(eh, av, vecs[1:2], vecs[2:3])
    ne = _dot(eh, mats[0:128]) + vecs[3:4]
    nh = u4 + _dot(ne, mats[128:256]) + vecs[4:5]
    nh = _lrelu_ln4(nh, av, vecs[5:6], vecs[6:7])
    no_ref[...] = _dot(nh, mats[256:384]) + vecs[7:8]
    ne_ref[...] = ne


_E4 = _EPAD // 4


def _tc_edge(gs4, gd4, ea4, we4, mats4, av, vecs4):
    de4 = ea4.shape[1]
    blk = 1024
    return pl.pallas_call(
        _edge_body,
        grid=(_E4 // blk,),
        in_specs=[
            pl.BlockSpec((blk, 256), lambda i: (i, 0)),
            pl.BlockSpec((blk, 128), lambda i: (i, 0)),
            pl.BlockSpec((blk, de4), lambda i: (i, 0)),
            pl.BlockSpec((de4, 128), lambda i: (0, 0)),
            pl.BlockSpec((384, 128), lambda i: (0, 0)),
            pl.BlockSpec((128, 128), lambda i: (0, 0)),
            pl.BlockSpec((8, 128), lambda i: (0, 0)),
        ],
        out_specs=(
            pl.BlockSpec((blk, 128), lambda i: (i, 0)),
            pl.BlockSpec((blk, 128), lambda i: (i, 0)),
        ),
        out_shape=(
            jax.ShapeDtypeStruct((_E4, 128), jnp.float32),
            jax.ShapeDtypeStruct((_E4, 128), jnp.float32),
        ),
    )(gs4, gd4, ea4, we4, mats4, av, vecs4)


def _node_body_tables(x_ref, p0_ref, p1_ref, c0_ref, c1_ref,
                      n21x_ref, mats_ref, vecs_ref, wsu_ref, wt_ref,
                      nx_ref, su_ref, t_ref):
    mats = mats_ref[...]
    vecs = vecs_ref[...]
    cnt = jnp.maximum(c0_ref[...] + c1_ref[...], 1.0)
    agg = (p0_ref[...] + p1_ref[...]) / cnt
    h = _dot(x_ref[...], n21x_ref[...]) + _dot(agg, mats[0:32]) + vecs[0:1]
    h = _lrelu_ln(h, vecs[1:2], vecs[2:3])
    nx = _dot(h, mats[32:64]) + vecs[3:4]
    nx_ref[...] = nx
    su_ref[...] = _dot(nx, wsu_ref[...])
    t_ref[...] = _dot(nx, wt_ref[...])


def _node_body_final(x_ref, p0_ref, p1_ref, c0_ref, c1_ref,
                     n21x_ref, mats_ref, vecs_ref, nx_ref):
    mats = mats_ref[...]
    vecs = vecs_ref[...]
    cnt = jnp.maximum(c0_ref[...] + c1_ref[...], 1.0)
    agg = (p0_ref[...] + p1_ref[...]) / cnt
    h = _dot(x_ref[...], n21x_ref[...]) + _dot(agg, mats[0:32]) + vecs[0:1]
    h = _lrelu_ln(h, vecs[1:2], vecs[2:3])
    nx_ref[...] = _dot(h, mats[32:64]) + vecs[3:4]


def _tc_node(x, p0, p1, c0, c1, n21x, mats, vecs, wsu=None, wt=None):
    d = x.shape[1]
    blk = 1024
    grid = (_NPAD // blk,)
    row = lambda i: (i, 0)
    bcast = lambda shape: pl.BlockSpec(shape, lambda i: (0, 0))
    in_specs = [
        pl.BlockSpec((blk, d), row),
        pl.BlockSpec((blk, 32), row),
        pl.BlockSpec((blk, 32), row),
        pl.BlockSpec((blk, 32), row),
        pl.BlockSpec((blk, 32), row),
        bcast((d, 32)),
        bcast((64, 32)),
        bcast((5, 32)),
    ]
    if wsu is not None:
        in_specs += [bcast((32, 64)), bcast((32, 32))]
        return pl.pallas_call(
            _node_body_tables,
            grid=grid,
            in_specs=in_specs,
            out_specs=(
                pl.BlockSpec((blk, 32), row),
                pl.BlockSpec((blk, 64), row),
                pl.BlockSpec((blk, 32), row),
            ),
            out_shape=(
                jax.ShapeDtypeStruct((_NPAD, 32), jnp.float32),
                jax.ShapeDtypeStruct((_NPAD, 64), jnp.float32),
                jax.ShapeDtypeStruct((_NPAD, 32), jnp.float32),
            ),
        )(x, p0, p1, c0, c1, n21x, mats, vecs, wsu, wt)
    return pl.pallas_call(
        _node_body_final,
        grid=grid,
        in_specs=in_specs,
        out_specs=pl.BlockSpec((blk, 32), row),
        out_shape=jax.ShapeDtypeStruct((_NPAD, 32), jnp.float32),
    )(x, p0, p1, c0, c1, n21x, mats, vecs)


def _bd4(w):
    return jnp.kron(jnp.eye(4, dtype=w.dtype), w)


def _pack_edge_weights(p, d):
    we4 = _bd4(p["e1_w"][2 * d:])
    mats4 = jnp.concatenate(
        [_bd4(p["e2_w"]), _bd4(p["n11_w"][d:]), _bd4(p["n12_w"])], axis=0)
    vecs = jnp.stack([
        p["e1_b"], p["lne_g"], p["lne_b"], p["e2_b"],
        p["n11_b"], p["lnn1_g"], p["lnn1_b"], p["n12_b"],
    ], axis=0)
    return we4, mats4, jnp.tile(vecs, (1, 4))


def _pack_node_weights(p, d):
    n21x = p["n21_w"][:d]
    mats = jnp.concatenate([p["n21_w"][d:], p["n22_w"]], axis=0)
    vecs = jnp.stack([
        p["n21_b"], p["lnn2_g"], p["lnn2_b"], p["n22_b"],
        jnp.zeros_like(p["n22_b"]),
    ], axis=0)
    return n21x, mats, vecs


def _table_weights(p, d):
    wsu = jnp.concatenate([p["e1_w"][:d], p["n11_w"][:d]], axis=1)
    wt = p["e1_w"][d:2 * d]
    return wsu, wt


def kernel(x, edge_index, edge_attr, params):
    f32 = jnp.float32
    src = edge_index[0]
    dst = edge_index[1]
    src2d = jnp.concatenate(
        [src, jnp.zeros((_EPAD - _E,), jnp.int32)]).reshape(_EPAD // _K, _K)
    dst2d = jnp.concatenate(
        [dst, jnp.full((_EPAD - _E,), _N, jnp.int32)]).reshape(_EPAD // _K, _K)
    x_pad = jnp.concatenate([x, jnp.zeros((_NPAD - _N, _D), f32)], axis=0)
    ea = jnp.concatenate(
        [edge_attr, jnp.zeros((_EPAD - _E, edge_attr.shape[1]), f32)], axis=0)

    c0, c1 = _sc_count(dst2d)

    dims = [_D, _H, _H]
    av = jnp.kron(jnp.eye(4, dtype=f32), jnp.full((32, 32), 1.0 / 32, f32))
    wsu, wt = _table_weights(params["l0"], _D)
    su, t = _tc_tables(x_pad, wsu, wt)
    x_cur = x_pad
    ea4 = ea.reshape(_E4, 4 * ea.shape[1])
    for li in range(3):
        p = params["l%d" % li]
        d = dims[li]
        gs, gd = _sc_gather(su, t, src2d, dst2d)
        we4, emats4, evecs4 = _pack_edge_weights(p, d)
        ne4, no4 = _tc_edge(gs.reshape(_E4, 256), gd.reshape(_E4, 128),
                            ea4, we4, emats4, av, evecs4)
        p0, p1 = _sc_scatter(no4.reshape(_EPAD, 32), dst2d)
        n21x, nmats, nvecs = _pack_node_weights(p, d)
        if li < 2:
            wsu, wt = _table_weights(params["l%d" % (li + 1)], _H)
            x_cur, su, t = _tc_node(x_cur, p0, p1, c0, c1,
                                    n21x, nmats, nvecs, wsu, wt)
        else:
            x_cur = _tc_node(x_cur, p0, p1, c0, c1, n21x, nmats, nvecs)
        ea4 = ne4
    return x_cur[:_N]


# speedup vs baseline: 1.0036x; 1.0036x over previous
import functools

import jax
import jax.numpy as jnp
from jax import lax
from jax.experimental import pallas as pl
from jax.experimental.pallas import tpu as pltpu
from jax.experimental.pallas import tpu_sc as plsc

_N = 10000
_E = 320000
_D = 128
_H = 32

_NC = 2
_NS = 16
_NW = _NC * _NS
_K = 128
_NPAD = 10240
_EPAD = 327680
_PW = _EPAD // _NW
_KCH = _PW // _K
_RPT = _NPAD // _NS

def _mesh():
    return plsc.VectorSubcoreMesh(
        core_axis_name="c", subcore_axis_name="s",
        num_cores=_NC, num_subcores=_NS)


_U = 8
_G = _KCH // _U
_NBUF = 4
_KCH0 = 104
_KCH1 = 2 * _KCH - _KCH0
_KCHM = max(_KCH0, _KCH1)


@functools.cache
def _sc_gather_fn():
    return functools.partial(
        pl.kernel,
        out_type=(
            jax.ShapeDtypeStruct((_EPAD, 64), jnp.float32),
            jax.ShapeDtypeStruct((_EPAD, 32), jnp.float32),
        ),
        mesh=_mesh(),
        compiler_params=pltpu.CompilerParams(use_tc_tiling_on_sc=False),
        scratch_types=[
            pltpu.VMEM((_KCHM, _K), jnp.int32),
            pltpu.VMEM((_KCHM, _K), jnp.int32),
        ] + [pltpu.VMEM((_K, 64), jnp.float32) for _ in range(_NBUF)]
          + [pltpu.VMEM((_K, 32), jnp.float32) for _ in range(_NBUF)]
          + [pltpu.SemaphoreType.DMA for _ in range(2 * _NBUF)],
    )(_sc_gather_body)


def _sc_gather(su, t, src2d, dst2d):
    return _sc_gather_fn()(su, t, src2d, dst2d)


def _sc_gather_body(su_hbm, t_hbm, src_hbm, dst_hbm, gs_hbm, gd_hbm,
                    sidx, didx, *bufsem):
    c = lax.axis_index("c")
    s = lax.axis_index("s")
    gsb = bufsem[0:_NBUF]
    gdb = bufsem[_NBUF:2 * _NBUF]
    gsem = bufsem[2 * _NBUF:3 * _NBUF]
    ssem = bufsem[3 * _NBUF:4 * _NBUF]

    def gath(base, j, b):
        return (pltpu.async_copy(su_hbm.at[sidx.at[j]], gsb[b], gsem[b]),
                pltpu.async_copy(t_hbm.at[didx.at[j]], gdb[b], gsem[b]))

    def stor(base, j, b):
        return (pltpu.async_copy(gsb[b], gs_hbm.at[pl.ds(base + j * _K, _K)],
                                 ssem[b]),
                pltpu.async_copy(gdb[b], gd_hbm.at[pl.ds(base + j * _K, _K)],
                                 ssem[b]))

    def run(start_chunk, kch):
        pltpu.sync_copy(src_hbm.at[pl.ds(start_chunk, kch)],
                        sidx.at[pl.ds(0, kch)])
        pltpu.sync_copy(dst_hbm.at[pl.ds(start_chunk, kch)],
                        didx.at[pl.ds(0, kch)])
        base = start_chunk * _K

        def group(g, carry):
            c0 = g * _U
            gd_ = [None] * _U
            st = [None] * _U
            for b in range(_U):
                if b >= _NBUF:
                    st[b - _NBUF][0].wait()
                    st[b - _NBUF][1].wait()
                gd_[b] = gath(base, c0 + b, b % _NBUF)
                ib = b - (_NBUF - 1)
                if ib >= 0:
                    gd_[ib][0].wait()
                    gd_[ib][1].wait()
                    st[ib] = stor(base, c0 + ib, ib % _NBUF)
            for ib in range(_U - _NBUF + 1, _U):
                gd_[ib][0].wait()
                gd_[ib][1].wait()
                st[ib] = stor(base, c0 + ib, ib % _NBUF)
            for ib in range(_U - _NBUF, _U):
                st[ib][0].wait()
                st[ib][1].wait()
            return carry

        lax.fori_loop(0, kch // _U, group, 0)

    start = s * (2 * _KCH) + c * _KCH0

    @pl.when(c == 0)
    def _():
        run(start, _KCH0)

    @pl.when(c == 1)
    def _():
        run(start, _KCH1)


@functools.cache
def _sc_scatter_fn():
    return functools.partial(
        pl.kernel,
        out_type=(
            jax.ShapeDtypeStruct((_NPAD, 32), jnp.float32),
            jax.ShapeDtypeStruct((_NPAD, 32), jnp.float32),
        ),
        mesh=_mesh(),
        compiler_params=pltpu.CompilerParams(use_tc_tiling_on_sc=False),
        scratch_types=[
            pltpu.VMEM((_KCH, _K), jnp.int32),
            pltpu.VMEM((_K, 32), jnp.float32),
            pltpu.VMEM((_K, 32), jnp.float32),
            pltpu.VMEM_SHARED((_NPAD, 32), jnp.float32),
            pltpu.SemaphoreType.DMA,
            pltpu.SemaphoreType.DMA,
        ],
    )(_sc_scatter_body)


def _sc_scatter(vals, dst2d):
    return _sc_scatter_fn()(vals, dst2d)


def _zero_vbuf(vbuf, val=0.0):
    zv = jnp.full((16,), val, jnp.float32)

    def zrow(i, carry):
        vbuf[i, pl.ds(0, 16)] = zv
        vbuf[i, pl.ds(16, 16)] = zv
        return carry

    lax.fori_loop(0, _K, zrow, 0)


def _sc_scatter_body(vals_hbm, dst_hbm, out0_hbm, out1_hbm,
                     didx, vbuf0, vbuf1, shared, sem0, sem1):
    c = lax.axis_index("c")
    s = lax.axis_index("s")
    wid = s * _NC + c
    base = wid * _PW
    pltpu.sync_copy(dst_hbm.at[pl.ds(wid * _KCH, _KCH)], didx)

    _zero_vbuf(vbuf0)
    for i in range(_RPT // _K):
        pltpu.sync_copy(vbuf0, shared.at[pl.ds(s * _RPT + i * _K, _K)])
    plsc.subcore_barrier()

    bufs = ((vbuf0, sem0), (vbuf1, sem1))

    def group(g, carry):
        c0 = g * _U
        vb, sm = bufs[0]
        pend = pltpu.async_copy(vals_hbm.at[pl.ds(base + c0 * _K, _K)], vb, sm)
        for b in range(_U):
            j = c0 + b
            vb, _ = bufs[b % 2]
            nvb, nsm = bufs[(b + 1) % 2]
            nxt = None
            if b + 1 < _U:
                nxt = pltpu.async_copy(
                    vals_hbm.at[pl.ds(base + (j + 1) * _K, _K)], nvb, nsm)
            pend.wait()
            pltpu.sync_copy(vb, shared.at[didx.at[j]], add=True)
            pend = nxt
        return carry

    lax.fori_loop(0, _G, group, 0)
    plsc.subcore_barrier()

    @pl.when(c == 0)
    def _():
        pltpu.sync_copy(shared.at[pl.ds(s * _RPT, _RPT)],
                        out0_hbm.at[pl.ds(s * _RPT, _RPT)])

    @pl.when(c == 1)
    def _():
        pltpu.sync_copy(shared.at[pl.ds(s * _RPT, _RPT)],
                        out1_hbm.at[pl.ds(s * _RPT, _RPT)])


@functools.cache
def _sc_count_fn():
    return functools.partial(
        pl.kernel,
        out_type=(
            jax.ShapeDtypeStruct((_NPAD, 32), jnp.float32),
            jax.ShapeDtypeStruct((_NPAD, 32), jnp.float32),
        ),
        mesh=_mesh(),
        compiler_params=pltpu.CompilerParams(use_tc_tiling_on_sc=False),
        scratch_types=[
            pltpu.VMEM((_KCH, _K), jnp.int32),
            pltpu.VMEM((_K, 32), jnp.float32),
            pltpu.VMEM_SHARED((_NPAD, 32), jnp.float32),
        ],
    )(_sc_count_body)


def _sc_count(dst2d):
    return _sc_count_fn()(dst2d)


def _sc_count_body(dst_hbm, out0_hbm, out1_hbm, didx, vbuf, shared):
    c = lax.axis_index("c")
    s = lax.axis_index("s")
    wid = s * _NC + c
    pltpu.sync_copy(dst_hbm.at[pl.ds(wid * _KCH, _KCH)], didx)

    _zero_vbuf(vbuf)
    for i in range(_RPT // _K):
        pltpu.sync_copy(vbuf, shared.at[pl.ds(s * _RPT + i * _K, _K)])
    plsc.subcore_barrier()
    _zero_vbuf(vbuf, 1.0)

    def step(j, carry):
        pltpu.sync_copy(vbuf, shared.at[didx.at[j]], add=True)
        return carry

    lax.fori_loop(0, _KCH, step, 0)
    plsc.subcore_barrier()

    @pl.when(c == 0)
    def _():
        pltpu.sync_copy(shared.at[pl.ds(s * _RPT, _RPT)],
                        out0_hbm.at[pl.ds(s * _RPT, _RPT)])

    @pl.when(c == 1)
    def _():
        pltpu.sync_copy(shared.at[pl.ds(s * _RPT, _RPT)],
                        out1_hbm.at[pl.ds(s * _RPT, _RPT)])


def _lrelu_ln(h, g, b):
    a = jnp.where(h >= 0, h, 0.01 * h)
    m = jnp.mean(a, axis=-1, keepdims=True)
    v = jnp.mean((a - m) ** 2, axis=-1, keepdims=True)
    return (a - m) * lax.rsqrt(v + 1e-5) * g + b


def _dot(a, b):
    return jnp.dot(a, b, preferred_element_type=jnp.float32)


def _tables_body(x_ref, wsu_ref, wt_ref, su_ref, t_ref):
    x = x_ref[...]
    su_ref[...] = _dot(x, wsu_ref[...])
    t_ref[...] = _dot(x, wt_ref[...])


def _tc_tables(x, wsu, wt):
    n, d = x.shape
    blk = 2048
    return pl.pallas_call(
        _tables_body,
        grid=(n // blk,),
        in_specs=[
            pl.BlockSpec((blk, d), lambda i: (i, 0)),
            pl.BlockSpec((d, 64), lambda i: (0, 0)),
            pl.BlockSpec((d, 32), lambda i: (0, 0)),
        ],
        out_specs=(
            pl.BlockSpec((blk, 64), lambda i: (i, 0)),
            pl.BlockSpec((blk, 32), lambda i: (i, 0)),
        ),
        out_shape=(
            jax.ShapeDtypeStruct((n, 64), jnp.float32),
            jax.ShapeDtypeStruct((n, 32), jnp.float32),
        ),
    )(x, wsu, wt)


def _lrelu_ln4(h, av, g, b):
    a = jnp.where(h >= 0, h, 0.01 * h)
    m = _dot(a, av)
    d = a - m
    v = _dot(d * d, av)
    return d * lax.rsqrt(v + 1e-5) * g + b


def _edge_body(gs_ref, gd_ref, ea_ref, we_ref, mats_ref, av_ref, vecs_ref,
               ne_ref, no_ref):
    x = gs_ref[...]
    s4 = jnp.concatenate(
        [x[:, 0:32], x[:, 64:96], x[:, 128:160], x[:, 192:224]], axis=1)
    u4 = jnp.concatenate(
        [x[:, 32:64], x[:, 96:128], x[:, 160:192], x[:, 224:256]], axis=1)
    av = av_ref[...]
    mats = mats_ref[...]
    vecs = vecs_ref[...]
    eh = s4 + gd_ref[...] + _dot(ea_ref[...], we_ref[...]) + vecs[0:1]
    eh = _lrelu_ln4(eh, av, vecs[1:2], vecs[2:3])
    ne = _dot(eh, mats[0:128]) + vecs[3:4]
    nh = u4 + _dot(ne, mats[128:256]) + vecs[4:5]
    nh = _lrelu_ln4(nh, av, vecs[5:6], vecs[6:7])
    no_ref[...] = _dot(nh, mats[256:384]) + vecs[7:8]
    ne_ref[...] = ne


_E4 = _EPAD // 4


def _tc_edge(gs4, gd4, ea4, we4, mats4, av, vecs4):
    de4 = ea4.shape[1]
    blk = 1024
    return pl.pallas_call(
        _edge_body,
        grid=(_E4 // blk,),
        in_specs=[
            pl.BlockSpec((blk, 256), lambda i: (i, 0)),
            pl.BlockSpec((blk, 128), lambda i: (i, 0)),
            pl.BlockSpec((blk, de4), lambda i: (i, 0)),
            pl.BlockSpec((de4, 128), lambda i: (0, 0)),
            pl.BlockSpec((384, 128), lambda i: (0, 0)),
            pl.BlockSpec((128, 128), lambda i: (0, 0)),
            pl.BlockSpec((8, 128), lambda i: (0, 0)),
        ],
        out_specs=(
            pl.BlockSpec((blk, 128), lambda i: (i, 0)),
            pl.BlockSpec((blk, 128), lambda i: (i, 0)),
        ),
        out_shape=(
            jax.ShapeDtypeStruct((_E4, 128), jnp.float32),
            jax.ShapeDtypeStruct((_E4, 128), jnp.float32),
        ),
    )(gs4, gd4, ea4, we4, mats4, av, vecs4)


def _node_body_tables(x_ref, p0_ref, p1_ref, c0_ref, c1_ref,
                      n21x_ref, mats_ref, vecs_ref, wsu_ref, wt_ref,
                      nx_ref, su_ref, t_ref):
    mats = mats_ref[...]
    vecs = vecs_ref[...]
    cnt = jnp.maximum(c0_ref[...] + c1_ref[...], 1.0)
    agg = (p0_ref[...] + p1_ref[...]) / cnt
    h = _dot(x_ref[...], n21x_ref[...]) + _dot(agg, mats[0:32]) + vecs[0:1]
    h = _lrelu_ln(h, vecs[1:2], vecs[2:3])
    nx = _dot(h, mats[32:64]) + vecs[3:4]
    nx_ref[...] = nx
    su_ref[...] = _dot(nx, wsu_ref[...])
    t_ref[...] = _dot(nx, wt_ref[...])


def _node_body_final(x_ref, p0_ref, p1_ref, c0_ref, c1_ref,
                     n21x_ref, mats_ref, vecs_ref, nx_ref):
    mats = mats_ref[...]
    vecs = vecs_ref[...]
    cnt = jnp.maximum(c0_ref[...] + c1_ref[...], 1.0)
    agg = (p0_ref[...] + p1_ref[...]) / cnt
    h = _dot(x_ref[...], n21x_ref[...]) + _dot(agg, mats[0:32]) + vecs[0:1]
    h = _lrelu_ln(h, vecs[1:2], vecs[2:3])
    nx_ref[...] = _dot(h, mats[32:64]) + vecs[3:4]


def _tc_node(x, p0, p1, c0, c1, n21x, mats, vecs, wsu=None, wt=None):
    d = x.shape[1]
    blk = 1024
    grid = (_NPAD // blk,)
    row = lambda i: (i, 0)
    bcast = lambda shape: pl.BlockSpec(shape, lambda i: (0, 0))
    in_specs = [
        pl.BlockSpec((blk, d), row),
        pl.BlockSpec((blk, 32), row),
        pl.BlockSpec((blk, 32), row),
        pl.BlockSpec((blk, 32), row),
        pl.BlockSpec((blk, 32), row),
        bcast((d, 32)),
        bcast((64, 32)),
        bcast((5, 32)),
    ]
    if wsu is not None:
        in_specs += [bcast((32, 64)), bcast((32, 32))]
        return pl.pallas_call(
            _node_body_tables,
            grid=grid,
            in_specs=in_specs,
            out_specs=(
                pl.BlockSpec((blk, 32), row),
                pl.BlockSpec((blk, 64), row),
                pl.BlockSpec((blk, 32), row),
            ),
            out_shape=(
                jax.ShapeDtypeStruct((_NPAD, 32), jnp.float32),
                jax.ShapeDtypeStruct((_NPAD, 64), jnp.float32),
                jax.ShapeDtypeStruct((_NPAD, 32), jnp.float32),
            ),
        )(x, p0, p1, c0, c1, n21x, mats, vecs, wsu, wt)
    return pl.pallas_call(
        _node_body_final,
        grid=grid,
        in_specs=in_specs,
        out_specs=pl.BlockSpec((blk, 32), row),
        out_shape=jax.ShapeDtypeStruct((_NPAD, 32), jnp.float32),
    )(x, p0, p1, c0, c1, n21x, mats, vecs)


def _bd4(w):
    return jnp.kron(jnp.eye(4, dtype=w.dtype), w)


def _pack_edge_weights(p, d):
    we4 = _bd4(p["e1_w"][2 * d:])
    mats4 = jnp.concatenate(
        [_bd4(p["e2_w"]), _bd4(p["n11_w"][d:]), _bd4(p["n12_w"])], axis=0)
    vecs = jnp.stack([
        p["e1_b"], p["lne_g"], p["lne_b"], p["e2_b"],
        p["n11_b"], p["lnn1_g"], p["lnn1_b"], p["n12_b"],
    ], axis=0)
    return we4, mats4, jnp.tile(vecs, (1, 4))


def _pack_node_weights(p, d):
    n21x = p["n21_w"][:d]
    mats = jnp.concatenate([p["n21_w"][d:], p["n22_w"]], axis=0)
    vecs = jnp.stack([
        p["n21_b"], p["lnn2_g"], p["lnn2_b"], p["n22_b"],
        jnp.zeros_like(p["n22_b"]),
    ], axis=0)
    return n21x, mats, vecs


def _table_weights(p, d):
    wsu = jnp.concatenate([p["e1_w"][:d], p["n11_w"][:d]], axis=1)
    wt = p["e1_w"][d:2 * d]
    return wsu, wt


def kernel(x, edge_index, edge_attr, params):
    f32 = jnp.float32
    src = edge_index[0]
    dst = edge_index[1]
    src2d = jnp.concatenate(
        [src, jnp.zeros((_EPAD - _E,), jnp.int32)]).reshape(_EPAD // _K, _K)
    dst2d = jnp.concatenate(
        [dst, jnp.full((_EPAD - _E,), _N, jnp.int32)]).reshape(_EPAD // _K, _K)
    x_pad = jnp.concatenate([x, jnp.zeros((_NPAD - _N, _D), f32)], axis=0)
    ea = jnp.concatenate(
        [edge_attr, jnp.zeros((_EPAD - _E, edge_attr.shape[1]), f32)], axis=0)

    c0, c1 = _sc_count(dst2d)

    dims = [_D, _H, _H]
    av = jnp.kron(jnp.eye(4, dtype=f32), jnp.full((32, 32), 1.0 / 32, f32))
    wsu, wt = _table_weights(params["l0"], _D)
    su, t = _tc_tables(x_pad, wsu, wt)
    x_cur = x_pad
    ea4 = ea.reshape(_E4, 4 * ea.shape[1])
    for li in range(3):
        p = params["l%d" % li]
        d = dims[li]
        gs, gd = _sc_gather(su, t, src2d, dst2d)
        we4, emats4, evecs4 = _pack_edge_weights(p, d)
        ne4, no4 = _tc_edge(gs.reshape(_E4, 256), gd.reshape(_E4, 128),
                            ea4, we4, emats4, av, evecs4)
        p0, p1 = _sc_scatter(no4.reshape(_EPAD, 32), dst2d)
        n21x, nmats, nvecs = _pack_node_weights(p, d)
        if li < 2:
            wsu, wt = _table_weights(params["l%d" % (li + 1)], _H)
            x_cur, su, t = _tc_node(x_cur, p0, p1, c0, c1,
                                    n21x, nmats, nvecs, wsu, wt)
        else:
            x_cur = _tc_node(x_cur, p0, p1, c0, c1, n21x, nmats, nvecs)
        ea4 = ne4
    return x_cur[:_N]


# speedup vs baseline: 1.0238x; 1.0201x over previous
import functools

import jax
import jax.numpy as jnp
from jax import lax
from jax.experimental import pallas as pl
from jax.experimental.pallas import tpu as pltpu
from jax.experimental.pallas import tpu_sc as plsc

_N = 10000
_E = 320000
_D = 128
_H = 32

_NC = 2
_NS = 16
_NW = _NC * _NS
_K = 128
_NPAD = 10240
_EPAD = 327680
_PW = _EPAD // _NW
_KCH = _PW // _K
_RPT = _NPAD // _NS

def _mesh():
    return plsc.VectorSubcoreMesh(
        core_axis_name="c", subcore_axis_name="s",
        num_cores=_NC, num_subcores=_NS)


_U = 8
_NBUF = 4


@functools.cache
def _sc_gather_fn(kch):
    return functools.partial(
        pl.kernel,
        out_type=(
            jax.ShapeDtypeStruct((_NW * kch * _K, 64), jnp.float32),
            jax.ShapeDtypeStruct((_NW * kch * _K, 32), jnp.float32),
        ),
        mesh=_mesh(),
        compiler_params=pltpu.CompilerParams(use_tc_tiling_on_sc=False),
        scratch_types=[
            pltpu.VMEM((kch, _K), jnp.int32),
            pltpu.VMEM((kch, _K), jnp.int32),
        ] + [pltpu.VMEM((_K, 64), jnp.float32) for _ in range(_NBUF)]
          + [pltpu.VMEM((_K, 32), jnp.float32) for _ in range(_NBUF)]
          + [pltpu.SemaphoreType.DMA for _ in range(2 * _NBUF)],
    )(functools.partial(_sc_gather_body, kch))


def _sc_gather(su, t, src2d, dst2d):
    return _sc_gather_fn(src2d.shape[0] // _NW)(su, t, src2d, dst2d)


def _sc_gather_body(kch, su_hbm, t_hbm, src_hbm, dst_hbm, gs_hbm, gd_hbm,
                    sidx, didx, *bufsem):
    wid = lax.axis_index("s") * _NC + lax.axis_index("c")
    gsb = bufsem[0:_NBUF]
    gdb = bufsem[_NBUF:2 * _NBUF]
    gsem = bufsem[2 * _NBUF:3 * _NBUF]
    ssem = bufsem[3 * _NBUF:4 * _NBUF]
    pltpu.sync_copy(src_hbm.at[pl.ds(wid * kch, kch)], sidx)
    pltpu.sync_copy(dst_hbm.at[pl.ds(wid * kch, kch)], didx)
    base = wid * kch * _K

    def gath(j, b):
        return (pltpu.async_copy(su_hbm.at[sidx.at[j]], gsb[b], gsem[b]),
                pltpu.async_copy(t_hbm.at[didx.at[j]], gdb[b], gsem[b]))

    def stor(j, b):
        return (pltpu.async_copy(gsb[b], gs_hbm.at[pl.ds(base + j * _K, _K)],
                                 ssem[b]),
                pltpu.async_copy(gdb[b], gd_hbm.at[pl.ds(base + j * _K, _K)],
                                 ssem[b]))

    def group(g, carry):
        c0 = g * _U
        gd_ = [None] * _U
        st = [None] * _U
        for b in range(_U):
            if b >= _NBUF:
                st[b - _NBUF][0].wait()
                st[b - _NBUF][1].wait()
            gd_[b] = gath(c0 + b, b % _NBUF)
            ib = b - (_NBUF - 1)
            if ib >= 0:
                gd_[ib][0].wait()
                gd_[ib][1].wait()
                st[ib] = stor(c0 + ib, ib % _NBUF)
        for ib in range(_U - _NBUF + 1, _U):
            gd_[ib][0].wait()
            gd_[ib][1].wait()
            st[ib] = stor(c0 + ib, ib % _NBUF)
        for ib in range(_U - _NBUF, _U):
            st[ib][0].wait()
            st[ib][1].wait()
        return carry

    lax.fori_loop(0, kch // _U, group, 0)


@functools.cache
def _sc_scatter_fn(kch):
    return functools.partial(
        pl.kernel,
        out_type=(
            jax.ShapeDtypeStruct((_NPAD, 32), jnp.float32),
            jax.ShapeDtypeStruct((_NPAD, 32), jnp.float32),
        ),
        mesh=_mesh(),
        compiler_params=pltpu.CompilerParams(use_tc_tiling_on_sc=False),
        scratch_types=[
            pltpu.VMEM((kch, _K), jnp.int32),
            pltpu.VMEM((_K, 32), jnp.float32),
            pltpu.VMEM((_K, 32), jnp.float32),
            pltpu.VMEM_SHARED((_NPAD, 32), jnp.float32),
            pltpu.SemaphoreType.DMA,
            pltpu.SemaphoreType.DMA,
        ],
    )(functools.partial(_sc_scatter_body, kch))


def _sc_scatter(vals, dst2d):
    return _sc_scatter_fn(dst2d.shape[0] // _NW)(vals, dst2d)


def _zero_vbuf(vbuf, val=0.0):
    zv = jnp.full((16,), val, jnp.float32)

    def zrow(i, carry):
        vbuf[i, pl.ds(0, 16)] = zv
        vbuf[i, pl.ds(16, 16)] = zv
        return carry

    lax.fori_loop(0, _K, zrow, 0)


def _sc_scatter_body(kch, vals_hbm, dst_hbm, out0_hbm, out1_hbm,
                     didx, vbuf0, vbuf1, shared, sem0, sem1):
    c = lax.axis_index("c")
    s = lax.axis_index("s")
    wid = s * _NC + c
    base = wid * kch * _K
    pltpu.sync_copy(dst_hbm.at[pl.ds(wid * kch, kch)], didx)

    _zero_vbuf(vbuf0)
    for i in range(_RPT // _K):
        pltpu.sync_copy(vbuf0, shared.at[pl.ds(s * _RPT + i * _K, _K)])
    plsc.subcore_barrier()

    bufs = ((vbuf0, sem0), (vbuf1, sem1))

    def group(g, carry):
        c0 = g * _U
        vb, sm = bufs[0]
        pend = pltpu.async_copy(vals_hbm.at[pl.ds(base + c0 * _K, _K)], vb, sm)
        for b in range(_U):
            j = c0 + b
            vb, _ = bufs[b % 2]
            nvb, nsm = bufs[(b + 1) % 2]
            nxt = None
            if b + 1 < _U:
                nxt = pltpu.async_copy(
                    vals_hbm.at[pl.ds(base + (j + 1) * _K, _K)], nvb, nsm)
            pend.wait()
            pltpu.sync_copy(vb, shared.at[didx.at[j]], add=True)
            pend = nxt
        return carry

    lax.fori_loop(0, kch // _U, group, 0)
    plsc.subcore_barrier()

    @pl.when(c == 0)
    def _():
        pltpu.sync_copy(shared.at[pl.ds(s * _RPT, _RPT)],
                        out0_hbm.at[pl.ds(s * _RPT, _RPT)])

    @pl.when(c == 1)
    def _():
        pltpu.sync_copy(shared.at[pl.ds(s * _RPT, _RPT)],
                        out1_hbm.at[pl.ds(s * _RPT, _RPT)])


@functools.cache
def _sc_count_fn():
    return functools.partial(
        pl.kernel,
        out_type=(
            jax.ShapeDtypeStruct((_NPAD, 32), jnp.float32),
            jax.ShapeDtypeStruct((_NPAD, 32), jnp.float32),
        ),
        mesh=_mesh(),
        compiler_params=pltpu.CompilerParams(use_tc_tiling_on_sc=False),
        scratch_types=[
            pltpu.VMEM((_KCH, _K), jnp.int32),
            pltpu.VMEM((_K, 32), jnp.float32),
            pltpu.VMEM_SHARED((_NPAD, 32), jnp.float32),
        ],
    )(_sc_count_body)


def _sc_count(dst2d):
    return _sc_count_fn()(dst2d)


def _sc_count_body(dst_hbm, out0_hbm, out1_hbm, didx, vbuf, shared):
    c = lax.axis_index("c")
    s = lax.axis_index("s")
    wid = s * _NC + c
    pltpu.sync_copy(dst_hbm.at[pl.ds(wid * _KCH, _KCH)], didx)

    _zero_vbuf(vbuf)
    for i in range(_RPT // _K):
        pltpu.sync_copy(vbuf, shared.at[pl.ds(s * _RPT + i * _K, _K)])
    plsc.subcore_barrier()
    _zero_vbuf(vbuf, 1.0)

    def step(j, carry):
        pltpu.sync_copy(vbuf, shared.at[didx.at[j]], add=True)
        return carry

    lax.fori_loop(0, _KCH, step, 0)
    plsc.subcore_barrier()

    @pl.when(c == 0)
    def _():
        pltpu.sync_copy(shared.at[pl.ds(s * _RPT, _RPT)],
                        out0_hbm.at[pl.ds(s * _RPT, _RPT)])

    @pl.when(c == 1)
    def _():
        pltpu.sync_copy(shared.at[pl.ds(s * _RPT, _RPT)],
                        out1_hbm.at[pl.ds(s * _RPT, _RPT)])


def _lrelu_ln(h, g, b):
    a = jnp.where(h >= 0, h, 0.01 * h)
    m = jnp.mean(a, axis=-1, keepdims=True)
    v = jnp.mean((a - m) ** 2, axis=-1, keepdims=True)
    return (a - m) * lax.rsqrt(v + 1e-5) * g + b


def _dot(a, b):
    return jnp.dot(a, b, preferred_element_type=jnp.float32)


def _tables_body(x_ref, wsu_ref, wt_ref, su_ref, t_ref):
    x = x_ref[...]
    su_ref[...] = _dot(x, wsu_ref[...])
    t_ref[...] = _dot(x, wt_ref[...])


def _tc_tables(x, wsu, wt):
    n, d = x.shape
    blk = 2048
    return pl.pallas_call(
        _tables_body,
        grid=(n // blk,),
        in_specs=[
            pl.BlockSpec((blk, d), lambda i: (i, 0)),
            pl.BlockSpec((d, 64), lambda i: (0, 0)),
            pl.BlockSpec((d, 32), lambda i: (0, 0)),
        ],
        out_specs=(
            pl.BlockSpec((blk, 64), lambda i: (i, 0)),
            pl.BlockSpec((blk, 32), lambda i: (i, 0)),
        ),
        out_shape=(
            jax.ShapeDtypeStruct((n, 64), jnp.float32),
            jax.ShapeDtypeStruct((n, 32), jnp.float32),
        ),
    )(x, wsu, wt)


def _lrelu_ln4(h, av, g, b):
    a = jnp.where(h >= 0, h, 0.01 * h)
    m = _dot(a, av)
    d = a - m
    v = _dot(d * d, av)
    return d * lax.rsqrt(v + 1e-5) * g + b


def _edge_body(gs_ref, gd_ref, ea_ref, we_ref, mats_ref, av_ref, vecs_ref,
               ne_ref, no_ref):
    x = gs_ref[...]
    s4 = jnp.concatenate(
        [x[:, 0:32], x[:, 64:96], x[:, 128:160], x[:, 192:224]], axis=1)
    u4 = jnp.concatenate(
        [x[:, 32:64], x[:, 96:128], x[:, 160:192], x[:, 224:256]], axis=1)
    av = av_ref[...]
    mats = mats_ref[...]
    vecs = vecs_ref[...]
    eh = s4 + gd_ref[...] + _dot(ea_ref[...], we_ref[...]) + vecs[0:1]
    eh = _lrelu_ln4(eh, av, vecs[1:2], vecs[2:3])
    ne = _dot(eh, mats[0:128]) + vecs[3:4]
    nh = u4 + _dot(ne, mats[128:256]) + vecs[4:5]
    nh = _lrelu_ln4(nh, av, vecs[5:6], vecs[6:7])
    no_ref[...] = _dot(nh, mats[256:384]) + vecs[7:8]
    ne_ref[...] = ne


_E4 = _EPAD // 4


def _tc_edge(gs4, gd4, ea4, we4, mats4, av, vecs4):
    de4 = ea4.shape[1]
    n4 = gs4.shape[0]
    blk = 1024
    return pl.pallas_call(
        _edge_body,
        grid=(n4 // blk,),
        in_specs=[
            pl.BlockSpec((blk, 256), lambda i: (i, 0)),
            pl.BlockSpec((blk, 128), lambda i: (i, 0)),
            pl.BlockSpec((blk, de4), lambda i: (i, 0)),
            pl.BlockSpec((de4, 128), lambda i: (0, 0)),
            pl.BlockSpec((384, 128), lambda i: (0, 0)),
            pl.BlockSpec((128, 128), lambda i: (0, 0)),
            pl.BlockSpec((8, 128), lambda i: (0, 0)),
        ],
        out_specs=(
            pl.BlockSpec((blk, 128), lambda i: (i, 0)),
            pl.BlockSpec((blk, 128), lambda i: (i, 0)),
        ),
        out_shape=(
            jax.ShapeDtypeStruct((n4, 128), jnp.float32),
            jax.ShapeDtypeStruct((n4, 128), jnp.float32),
        ),
    )(gs4, gd4, ea4, we4, mats4, av, vecs4)


def _node_body_tables(x_ref, p0_ref, p1_ref, p2_ref, p3_ref, c0_ref, c1_ref,
                      n21x_ref, mats_ref, vecs_ref, wsu_ref, wt_ref,
                      nx_ref, su_ref, t_ref):
    mats = mats_ref[...]
    vecs = vecs_ref[...]
    cnt = jnp.maximum(c0_ref[...] + c1_ref[...], 1.0)
    agg = (p0_ref[...] + p1_ref[...] + p2_ref[...] + p3_ref[...]) / cnt
    h = _dot(x_ref[...], n21x_ref[...]) + _dot(agg, mats[0:32]) + vecs[0:1]
    h = _lrelu_ln(h, vecs[1:2], vecs[2:3])
    nx = _dot(h, mats[32:64]) + vecs[3:4]
    nx_ref[...] = nx
    su_ref[...] = _dot(nx, wsu_ref[...])
    t_ref[...] = _dot(nx, wt_ref[...])


def _node_body_final(x_ref, p0_ref, p1_ref, p2_ref, p3_ref, c0_ref, c1_ref,
                     n21x_ref, mats_ref, vecs_ref, nx_ref):
    mats = mats_ref[...]
    vecs = vecs_ref[...]
    cnt = jnp.maximum(c0_ref[...] + c1_ref[...], 1.0)
    agg = (p0_ref[...] + p1_ref[...] + p2_ref[...] + p3_ref[...]) / cnt
    h = _dot(x_ref[...], n21x_ref[...]) + _dot(agg, mats[0:32]) + vecs[0:1]
    h = _lrelu_ln(h, vecs[1:2], vecs[2:3])
    nx_ref[...] = _dot(h, mats[32:64]) + vecs[3:4]


def _tc_node(x, p0, p1, p2, p3, c0, c1, n21x, mats, vecs, wsu=None, wt=None):
    d = x.shape[1]
    blk = 1024
    grid = (_NPAD // blk,)
    row = lambda i: (i, 0)
    bcast = lambda shape: pl.BlockSpec(shape, lambda i: (0, 0))
    in_specs = [
        pl.BlockSpec((blk, d), row),
        pl.BlockSpec((blk, 32), row),
        pl.BlockSpec((blk, 32), row),
        pl.BlockSpec((blk, 32), row),
        pl.BlockSpec((blk, 32), row),
        pl.BlockSpec((blk, 32), row),
        pl.BlockSpec((blk, 32), row),
        bcast((d, 32)),
        bcast((64, 32)),
        bcast((5, 32)),
    ]
    if wsu is not None:
        in_specs += [bcast((32, 64)), bcast((32, 32))]
        return pl.pallas_call(
            _node_body_tables,
            grid=grid,
            in_specs=in_specs,
            out_specs=(
                pl.BlockSpec((blk, 32), row),
                pl.BlockSpec((blk, 64), row),
                pl.BlockSpec((blk, 32), row),
            ),
            out_shape=(
                jax.ShapeDtypeStruct((_NPAD, 32), jnp.float32),
                jax.ShapeDtypeStruct((_NPAD, 64), jnp.float32),
                jax.ShapeDtypeStruct((_NPAD, 32), jnp.float32),
            ),
        )(x, p0, p1, p2, p3, c0, c1, n21x, mats, vecs, wsu, wt)
    return pl.pallas_call(
        _node_body_final,
        grid=grid,
        in_specs=in_specs,
        out_specs=pl.BlockSpec((blk, 32), row),
        out_shape=jax.ShapeDtypeStruct((_NPAD, 32), jnp.float32),
    )(x, p0, p1, p2, p3, c0, c1, n21x, mats, vecs)


def _bd4(w):
    return jnp.kron(jnp.eye(4, dtype=w.dtype), w)


def _pack_edge_weights(p, d):
    we4 = _bd4(p["e1_w"][2 * d:])
    mats4 = jnp.concatenate(
        [_bd4(p["e2_w"]), _bd4(p["n11_w"][d:]), _bd4(p["n12_w"])], axis=0)
    vecs = jnp.stack([
        p["e1_b"], p["lne_g"], p["lne_b"], p["e2_b"],
        p["n11_b"], p["lnn1_g"], p["lnn1_b"], p["n12_b"],
    ], axis=0)
    return we4, mats4, jnp.tile(vecs, (1, 4))


def _pack_node_weights(p, d):
    n21x = p["n21_w"][:d]
    mats = jnp.concatenate([p["n21_w"][d:], p["n22_w"]], axis=0)
    vecs = jnp.stack([
        p["n21_b"], p["lnn2_g"], p["lnn2_b"], p["n22_b"],
        jnp.zeros_like(p["n22_b"]),
    ], axis=0)
    return n21x, mats, vecs


def _table_weights(p, d):
    wsu = jnp.concatenate([p["e1_w"][:d], p["n11_w"][:d]], axis=1)
    wt = p["e1_w"][d:2 * d]
    return wsu, wt


def kernel(x, edge_index, edge_attr, params):
    f32 = jnp.float32
    src = edge_index[0]
    dst = edge_index[1]
    src2d = jnp.concatenate(
        [src, jnp.zeros((_EPAD - _E,), jnp.int32)]).reshape(_EPAD // _K, _K)
    dst2d = jnp.concatenate(
        [dst, jnp.full((_EPAD - _E,), _N, jnp.int32)]).reshape(_EPAD // _K, _K)
    x_pad = jnp.concatenate([x, jnp.zeros((_NPAD - _N, _D), f32)], axis=0)
    ea = jnp.concatenate(
        [edge_attr, jnp.zeros((_EPAD - _E, edge_attr.shape[1]), f32)], axis=0)

    nch = _EPAD // _K
    src_h = [src2d[:nch // 2], src2d[nch // 2:]]
    dst_h = [dst2d[:nch // 2], dst2d[nch // 2:]]
    e4h = _E4 // 2
    ea4 = ea.reshape(_E4, 4 * ea.shape[1])
    ea4_h = [ea4[:e4h], ea4[e4h:]]

    c0, c1 = _sc_count(dst2d)

    dims = [_D, _H, _H]
    av = jnp.kron(jnp.eye(4, dtype=f32), jnp.full((32, 32), 1.0 / 32, f32))
    wsu, wt = _table_weights(params["l0"], _D)
    su, t = _tc_tables(x_pad, wsu, wt)
    x_cur = x_pad
    for li in range(3):
        p = params["l%d" % li]
        d = dims[li]
        we4, emats4, evecs4 = _pack_edge_weights(p, d)
        parts = []
        ne_h = []
        for h in range(2):
            gs, gd = _sc_gather(su, t, src_h[h], dst_h[h])
            ne4, no4 = _tc_edge(gs.reshape(e4h, 256), gd.reshape(e4h, 128),
                                ea4_h[h], we4, emats4, av, evecs4)
            ne_h.append(ne4)
            parts.extend(_sc_scatter(no4.reshape(_EPAD // 2, 32), dst_h[h]))
        ea4_h = ne_h
        n21x, nmats, nvecs = _pack_node_weights(p, d)
        if li < 2:
            wsu, wt = _table_weights(params["l%d" % (li + 1)], _H)
            x_cur, su, t = _tc_node(x_cur, parts[0], parts[1], parts[2],
                                    parts[3], c0, c1, n21x, nmats, nvecs,
                                    wsu, wt)
        else:
            x_cur = _tc_node(x_cur, parts[0], parts[1], parts[2], parts[3],
                             c0, c1, n21x, nmats, nvecs)
    return x_cur[:_N]
